# baseline (speedup 1.0000x reference)
import functools

import jax
import jax.numpy as jnp
from jax import lax
from jax.experimental import pallas as pl
from jax.experimental.pallas import tpu as pltpu
from jax.experimental.pallas import tpu_sc as plsc

_NC = 2
_NS = 16
_NW = _NC * _NS
_LANES = 16

_EPS = 1e-5
_ROWS = 4


def _rsqrt16(x):
    i = lax.bitcast_convert_type(x, jnp.int32)
    i = jnp.int32(0x5F3759DF) - lax.shift_right_logical(i, 1)
    y = lax.bitcast_convert_type(i, jnp.float32)
    half_x = 0.5 * x
    for _ in range(2):
        y = y * (1.5 - half_x * y * y)
    return y


def _lane_sum(x, perms):
    for perm in perms:
        x = x + x.at[perm].get(mode="promise_in_bounds")
    return x


def _tree_add(vs):
    while len(vs) > 1:
        vs = [a + b for a, b in zip(vs[::2], vs[1::2])]
    return vs[0]


def _make_kernel(L, T, C, B):
    D = T * C
    CL = C // 128
    per_w = B // _NW
    n_chunks = per_w // _ROWS
    sub = C // _LANES

    mesh = plsc.VectorSubcoreMesh(core_axis_name="c", subcore_axis_name="s")

    @functools.partial(
        pl.kernel,
        mesh=mesh,
        out_type=jax.ShapeDtypeStruct((B, T, C), jnp.float32),
        scratch_types=[
            pltpu.VMEM((n_chunks, _ROWS), jnp.int32),
            pltpu.VMEM((_ROWS, D // 128, 128), jnp.float32),
            pltpu.VMEM((_ROWS, D // 128, 128), jnp.float32),
            pltpu.VMEM((_ROWS, T, C), jnp.float32),
            pltpu.VMEM((_ROWS, T, C), jnp.float32),
            pltpu.VMEM((C,), jnp.float32),
            pltpu.VMEM((C,), jnp.float32),
            pltpu.SemaphoreType.DMA,
            pltpu.SemaphoreType.DMA,
            pltpu.SemaphoreType.DMA,
            pltpu.SemaphoreType.DMA,
        ],
    )
    def fused_kernel(tab_hbm, idx_hbm, gamma_hbm, beta_hbm, out_hbm,
                     idx_v, in0, in1, st0, st1, gam_v, bet_v,
                     si0, si1, so0, so1):
        wid = lax.axis_index("s") * _NC + lax.axis_index("c")
        base = wid * per_w
        pltpu.sync_copy(idx_hbm.at[wid], idx_v)
        pltpu.sync_copy(gamma_hbm, gam_v)
        pltpu.sync_copy(beta_hbm, bet_v)
        lane = lax.iota(jnp.int32, _LANES)
        perms = tuple(lane ^ step for step in (8, 4, 2, 1))
        inv_n = jnp.float32(1.0 / C)

        inb = (in0, in1)
        stb = (st0, st1)
        sin = (si0, si1)
        sout = (so0, so1)

        def compute(src, dst):
            def per_rpair(rp, carry):
                r0 = 2 * rp
                r1 = r0 + 1

                def per_quad(th, carry2):
                    u0 = (2 * th) * CL
                    u1 = u0 + CL
                    zeros = jnp.zeros((_LANES,), jnp.float32)
                    init = (zeros,) * 8

                    def acc(u, carry3):
                        a00, q00, a01, q01, a10, q10, a11, q11 = carry3
                        for k in range(8):
                            col = k * _LANES
                            v00 = src[r0, u0 + u, pl.ds(col, _LANES)]
                            v01 = src[r0, u1 + u, pl.ds(col, _LANES)]
                            v10 = src[r1, u0 + u, pl.ds(col, _LANES)]
                            v11 = src[r1, u1 + u, pl.ds(col, _LANES)]
                            a00 += v00
                            q00 += v00 * v00
                            a01 += v01
                            q01 += v01 * v01
                            a10 += v10
                            q10 += v10 * v10
                            a11 += v11
                            q11 += v11 * v11
                        return (a00, q00, a01, q01, a10, q10, a11, q11)

                    s00, q00, s01, q01, s10, q10, s11, q11 = lax.fori_loop(
                        0, CL, acc, init
                    )
                    m00 = _lane_sum(s00, perms) * inv_n
                    m01 = _lane_sum(s01, perms) * inv_n
                    m10 = _lane_sum(s10, perms) * inv_n
                    m11 = _lane_sum(s11, perms) * inv_n
                    r00 = _rsqrt16(_lane_sum(q00, perms) * inv_n - m00 * m00 + _EPS)
                    r01 = _rsqrt16(_lane_sum(q01, perms) * inv_n - m01 * m01 + _EPS)
                    r10 = _rsqrt16(_lane_sum(q10, perms) * inv_n - m10 * m10 + _EPS)
                    r11 = _rsqrt16(_lane_sum(q11, perms) * inv_n - m11 * m11 + _EPS)

                    def norm(u, carry3):
                        ubase = u * (8 * _LANES)
                        for k in range(8):
                            col = k * _LANES
                            off = ubase + col
                            g = gam_v[pl.ds(off, _LANES)]
                            b = bet_v[pl.ds(off, _LANES)]
                            v00 = src[r0, u0 + u, pl.ds(col, _LANES)]
                            v01 = src[r0, u1 + u, pl.ds(col, _LANES)]
                            v10 = src[r1, u0 + u, pl.ds(col, _LANES)]
                            v11 = src[r1, u1 + u, pl.ds(col, _LANES)]
                            dst[r0, 2 * th, pl.ds(off, _LANES)] = (v00 - m00) * r00 * g + b
                            dst[r0, 2 * th + 1, pl.ds(off, _LANES)] = (v01 - m01) * r01 * g + b
                            dst[r1, 2 * th, pl.ds(off, _LANES)] = (v10 - m10) * r10 * g + b
                            dst[r1, 2 * th + 1, pl.ds(off, _LANES)] = (v11 - m11) * r11 * g + b
                        return carry3

                    lax.fori_loop(0, CL, norm, 0)
                    return carry2

                return lax.fori_loop(0, T // 2, per_quad, carry)

            lax.fori_loop(0, _ROWS // 2, per_rpair, 0)

        def wait_in(k):
            pltpu.make_async_copy(tab_hbm.at[idx_v.at[0]], inb[k], sin[k]).wait()

        def wait_out(k):
            pltpu.make_async_copy(
                stb[k], out_hbm.at[pl.ds(0, _ROWS)], sout[k]
            ).wait()

        def chunk_pair(cbase, first, last):
            for k in (0, 1):
                c = cbase + k
                wait_in(k)
                if not first:
                    wait_out(k)
                compute(inb[k], stb[k])
                pltpu.async_copy(
                    stb[k], out_hbm.at[pl.ds(base + c * _ROWS, _ROWS)], sout[k]
                )
                if not last:
                    pltpu.async_copy(
                        tab_hbm.at[idx_v.at[c + 2]], inb[k], sin[k]
                    )

        pltpu.async_copy(tab_hbm.at[idx_v.at[0]], in0, si0)
        pltpu.async_copy(tab_hbm.at[idx_v.at[1]], in1, si1)

        chunk_pair(0, first=True, last=False)

        def body(i, carry):
            chunk_pair(2 * i, first=False, last=False)
            return carry

        lax.fori_loop(1, n_chunks // 2 - 1, body, 0)

        chunk_pair(n_chunks - 2, first=False, last=True)
        wait_out(0)
        wait_out(1)

    return fused_kernel


def kernel(instructions, gamma, beta, idx_subject, idx_label):
    S, L, T, C = instructions.shape
    B = idx_label.shape[0]

    tab = jnp.reshape(instructions[idx_subject], (L, (T * C) // 128, 128))
    per_w = B // _NW
    idx = jnp.reshape(idx_label.astype(jnp.int32), (_NW, per_w // _ROWS, _ROWS))
    fn = _make_kernel(L, T, C, B)
    return fn(tab, idx, gamma, beta)


# speedup vs baseline: 1.5992x; 1.5992x over previous
import functools

import jax
import jax.numpy as jnp
from jax import lax
from jax.experimental import pallas as pl
from jax.experimental.pallas import tpu as pltpu
from jax.experimental.pallas import tpu_sc as plsc

_NC = 2
_NS = 16
_NW = _NC * _NS
_LANES = 16

_EPS = 1e-5
_RC = 32
_BSUB = 4


def _rsqrt16(x):
    i = lax.bitcast_convert_type(x, jnp.int32)
    i = jnp.int32(0x5F3759DF) - lax.shift_right_logical(i, 1)
    y = lax.bitcast_convert_type(i, jnp.float32)
    half_x = 0.5 * x
    for _ in range(2):
        y = y * (1.5 - half_x * y * y)
    return y


def _lane_sum(x, perms):
    for perm in perms:
        x = x + x.at[perm].get(mode="promise_in_bounds")
    return x


def _make_kernel(L, T, C, B):
    per_w = B // _NW
    n_chunks = T * _BSUB
    sub8 = C // (8 * _LANES)
    assert per_w == _BSUB * _RC

    mesh = plsc.VectorSubcoreMesh(core_axis_name="c", subcore_axis_name="s")

    @functools.partial(
        pl.kernel,
        mesh=mesh,
        out_type=jax.ShapeDtypeStruct((T * B, C), jnp.float32),
        scratch_types=[
            pltpu.VMEM((n_chunks, _RC), jnp.int32),
            pltpu.VMEM((_RC, C), jnp.float32),
            pltpu.VMEM((_RC, C), jnp.float32),
            pltpu.VMEM((_RC, C), jnp.float32),
            pltpu.VMEM((_RC, C), jnp.float32),
            pltpu.VMEM((C,), jnp.float32),
            pltpu.VMEM((C,), jnp.float32),
            pltpu.SemaphoreType.DMA,
            pltpu.SemaphoreType.DMA,
            pltpu.SemaphoreType.DMA,
            pltpu.SemaphoreType.DMA,
        ],
    )
    def tm_kernel(tab_hbm, idx_hbm, gamma_hbm, beta_hbm, out_hbm,
                  idx_v, in0, in1, st0, st1, gam_v, bet_v,
                  si0, si1, so0, so1):
        wid = lax.axis_index("s") * _NC + lax.axis_index("c")
        base_b = wid * per_w
        pltpu.sync_copy(idx_hbm.at[wid], idx_v)
        pltpu.sync_copy(gamma_hbm, gam_v)
        pltpu.sync_copy(beta_hbm, bet_v)
        lane = lax.iota(jnp.int32, _LANES)
        perms = tuple(lane ^ step for step in (8, 4, 2, 1))
        inv_n = jnp.float32(1.0 / C)

        inb = (in0, in1)
        stb = (st0, st1)
        sin = (si0, si1)
        sout = (so0, so1)

        def compute(src, dst):
            def per_quad(q, carry):
                r0 = 4 * q
                r1 = r0 + 1
                r2 = r0 + 2
                r3 = r0 + 3
                zeros = jnp.zeros((_LANES,), jnp.float32)
                init = (zeros,) * 8

                def acc(u, carry3):
                    a0, q0, a1, q1, a2, q2, a3, q3 = carry3
                    ubase = u * (8 * _LANES)
                    for k in range(8):
                        col = ubase + k * _LANES
                        v0 = src[r0, pl.ds(col, _LANES)]
                        v1 = src[r1, pl.ds(col, _LANES)]
                        v2 = src[r2, pl.ds(col, _LANES)]
                        v3 = src[r3, pl.ds(col, _LANES)]
                        a0 += v0
                        q0 += v0 * v0
                        a1 += v1
                        q1 += v1 * v1
                        a2 += v2
                        q2 += v2 * v2
                        a3 += v3
                        q3 += v3 * v3
                    return (a0, q0, a1, q1, a2, q2, a3, q3)

                s0, q0, s1, q1, s2, q2, s3, q3 = lax.fori_loop(0, sub8, acc, init)
                m0 = _lane_sum(s0, perms) * inv_n
                m1 = _lane_sum(s1, perms) * inv_n
                m2 = _lane_sum(s2, perms) * inv_n
                m3 = _lane_sum(s3, perms) * inv_n
                c0 = _rsqrt16(_lane_sum(q0, perms) * inv_n - m0 * m0 + _EPS)
                c1 = _rsqrt16(_lane_sum(q1, perms) * inv_n - m1 * m1 + _EPS)
                c2 = _rsqrt16(_lane_sum(q2, perms) * inv_n - m2 * m2 + _EPS)
                c3 = _rsqrt16(_lane_sum(q3, perms) * inv_n - m3 * m3 + _EPS)

                def norm(u, carry3):
                    ubase = u * (8 * _LANES)
                    for k in range(8):
                        col = ubase + k * _LANES
                        g = gam_v[pl.ds(col, _LANES)]
                        b = bet_v[pl.ds(col, _LANES)]
                        v0 = src[r0, pl.ds(col, _LANES)]
                        v1 = src[r1, pl.ds(col, _LANES)]
                        v2 = src[r2, pl.ds(col, _LANES)]
                        v3 = src[r3, pl.ds(col, _LANES)]
                        dst[r0, pl.ds(col, _LANES)] = (v0 - m0) * c0 * g + b
                        dst[r1, pl.ds(col, _LANES)] = (v1 - m1) * c1 * g + b
                        dst[r2, pl.ds(col, _LANES)] = (v2 - m2) * c2 * g + b
                        dst[r3, pl.ds(col, _LANES)] = (v3 - m3) * c3 * g + b
                    return carry3

                lax.fori_loop(0, sub8, norm, 0)
                return carry

            lax.fori_loop(0, _RC // 4, per_quad, 0)

        def out_row0(c):
            return (c // _BSUB) * B + base_b + (c % _BSUB) * _RC

        def wait_in(k):
            pltpu.make_async_copy(tab_hbm.at[idx_v.at[0]], inb[k], sin[k]).wait()

        def wait_out(k):
            pltpu.make_async_copy(
                stb[k], out_hbm.at[pl.ds(0, _RC)], sout[k]
            ).wait()

        def chunk_pair(cbase, first, last):
            for k in (0, 1):
                c = cbase + k
                wait_in(k)
                if not first:
                    wait_out(k)
                compute(inb[k], stb[k])
                pltpu.async_copy(
                    stb[k], out_hbm.at[pl.ds(out_row0(c), _RC)], sout[k]
                )
                if not last:
                    pltpu.async_copy(
                        tab_hbm.at[idx_v.at[c + 2]], inb[k], sin[k]
                    )

        pltpu.async_copy(tab_hbm.at[idx_v.at[0]], in0, si0)
        pltpu.async_copy(tab_hbm.at[idx_v.at[1]], in1, si1)

        chunk_pair(0, first=True, last=False)

        def body(i, carry):
            chunk_pair(2 * i, first=False, last=False)
            return carry

        lax.fori_loop(1, n_chunks // 2 - 1, body, 0)

        chunk_pair(n_chunks - 2, first=False, last=True)
        wait_out(0)
        wait_out(1)

    return tm_kernel


def kernel(instructions, gamma, beta, idx_subject, idx_label):
    S, L, T, C = instructions.shape
    B = idx_label.shape[0]

    tab = jnp.transpose(instructions[idx_subject], (1, 0, 2)).reshape(T * L, C)

    idxc = idx_label.astype(jnp.int32).reshape(_NW, _BSUB, _RC)
    gidx = (
        (jnp.arange(T, dtype=jnp.int32) * L)[None, :, None, None]
        + idxc[:, None, :, :]
    ).reshape(_NW, T * _BSUB, _RC)

    fn = _make_kernel(L, T, C, B)
    out = fn(tab, gidx, gamma, beta)

    return jnp.transpose(out.reshape(T, B, C), (1, 0, 2))


# speedup vs baseline: 1.7228x; 1.0773x over previous
import functools

import jax
import jax.numpy as jnp
from jax import lax
from jax.experimental import pallas as pl
from jax.experimental.pallas import tpu as pltpu
from jax.experimental.pallas import tpu_sc as plsc

_NC = 2
_NS = 16
_NW = _NC * _NS
_LANES = 16

_EPS = 1e-5
_RC = 32
_BSUB = 4


def _rsqrt16(x):
    i = lax.bitcast_convert_type(x, jnp.int32)
    i = jnp.int32(0x5F3759DF) - lax.shift_right_logical(i, 1)
    y = lax.bitcast_convert_type(i, jnp.float32)
    half_x = 0.5 * x
    for _ in range(2):
        y = y * (1.5 - half_x * y * y)
    return y


def _lane_sum(x, perms):
    for perm in perms:
        x = x + x.at[perm].get(mode="promise_in_bounds")
    return x


def _make_kernel(L, T, C, B):
    per_w = B // _NW
    n_chunks = T * _BSUB
    sub8 = C // (8 * _LANES)
    assert per_w == _BSUB * _RC

    mesh = plsc.VectorSubcoreMesh(core_axis_name="c", subcore_axis_name="s")

    @functools.partial(
        pl.kernel,
        mesh=mesh,
        out_type=jax.ShapeDtypeStruct((T * B, C), jnp.float32),
        scratch_types=[
            pltpu.VMEM((n_chunks, _RC), jnp.int32),
            pltpu.VMEM((_RC, C), jnp.float32),
            pltpu.VMEM((_RC, C), jnp.float32),
            pltpu.VMEM((_RC, C), jnp.float32),
            pltpu.VMEM((_RC, C), jnp.float32),
            pltpu.VMEM((C,), jnp.float32),
            pltpu.VMEM((C,), jnp.float32),
            pltpu.SemaphoreType.DMA,
            pltpu.SemaphoreType.DMA,
            pltpu.SemaphoreType.DMA,
            pltpu.SemaphoreType.DMA,
        ],
    )
    def tm_kernel(tab_hbm, idx_hbm, gamma_hbm, beta_hbm, out_hbm,
                  idx_v, in0, in1, st0, st1, gam_v, bet_v,
                  si0, si1, so0, so1):
        wid = lax.axis_index("s") * _NC + lax.axis_index("c")
        base_b = wid * per_w
        pltpu.sync_copy(idx_hbm.at[wid], idx_v)
        pltpu.sync_copy(gamma_hbm, gam_v)
        pltpu.sync_copy(beta_hbm, bet_v)
        lane = lax.iota(jnp.int32, _LANES)
        perms = tuple(lane ^ step for step in (8, 4, 2, 1))
        inv_n = jnp.float32(1.0 / C)

        inb = (in0, in1)
        stb = (st0, st1)
        sin = (si0, si1)
        sout = (so0, so1)

        def compute(src, dst):
            def per_quad(q, carry):
                r0 = 4 * q
                r1 = r0 + 1
                r2 = r0 + 2
                r3 = r0 + 3
                zeros = jnp.zeros((_LANES,), jnp.float32)
                init = (zeros,) * 8

                def acc(u, carry3):
                    a0, q0, a1, q1, a2, q2, a3, q3 = carry3
                    ubase = u * (8 * _LANES)
                    for k in range(8):
                        col = ubase + k * _LANES
                        v0 = src[r0, pl.ds(col, _LANES)]
                        v1 = src[r1, pl.ds(col, _LANES)]
                        v2 = src[r2, pl.ds(col, _LANES)]
                        v3 = src[r3, pl.ds(col, _LANES)]
                        a0 += v0
                        q0 += v0 * v0
                        a1 += v1
                        q1 += v1 * v1
                        a2 += v2
                        q2 += v2 * v2
                        a3 += v3
                        q3 += v3 * v3
                    return (a0, q0, a1, q1, a2, q2, a3, q3)

                s0, q0, s1, q1, s2, q2, s3, q3 = lax.fori_loop(0, sub8, acc, init)
                m0 = _lane_sum(s0, perms) * inv_n
                m1 = _lane_sum(s1, perms) * inv_n
                m2 = _lane_sum(s2, perms) * inv_n
                m3 = _lane_sum(s3, perms) * inv_n
                c0 = _rsqrt16(_lane_sum(q0, perms) * inv_n - m0 * m0 + _EPS)
                c1 = _rsqrt16(_lane_sum(q1, perms) * inv_n - m1 * m1 + _EPS)
                c2 = _rsqrt16(_lane_sum(q2, perms) * inv_n - m2 * m2 + _EPS)
                c3 = _rsqrt16(_lane_sum(q3, perms) * inv_n - m3 * m3 + _EPS)

                def norm(u, carry3):
                    ubase = u * (8 * _LANES)
                    for k in range(8):
                        col = ubase + k * _LANES
                        g = gam_v[pl.ds(col, _LANES)]
                        b = bet_v[pl.ds(col, _LANES)]
                        v0 = src[r0, pl.ds(col, _LANES)]
                        v1 = src[r1, pl.ds(col, _LANES)]
                        v2 = src[r2, pl.ds(col, _LANES)]
                        v3 = src[r3, pl.ds(col, _LANES)]
                        dst[r0, pl.ds(col, _LANES)] = (v0 - m0) * c0 * g + b
                        dst[r1, pl.ds(col, _LANES)] = (v1 - m1) * c1 * g + b
                        dst[r2, pl.ds(col, _LANES)] = (v2 - m2) * c2 * g + b
                        dst[r3, pl.ds(col, _LANES)] = (v3 - m3) * c3 * g + b
                    return carry3

                lax.fori_loop(0, sub8, norm, 0)
                return carry

            lax.fori_loop(0, _RC // 4, per_quad, 0)

        def out_row0(c):
            return (c // _BSUB) * B + base_b + (c % _BSUB) * _RC

        def wait_in(k):
            pltpu.make_async_copy(tab_hbm.at[idx_v.at[0]], inb[k], sin[k]).wait()

        def wait_out(k):
            pltpu.make_async_copy(
                stb[k], out_hbm.at[pl.ds(0, _RC)], sout[k]
            ).wait()

        def chunk_pair(cbase, first, last):
            for k in (0, 1):
                c = cbase + k
                wait_in(k)
                if not first:
                    wait_out(k)
                compute(inb[k], stb[k])
                pltpu.async_copy(
                    stb[k], out_hbm.at[pl.ds(out_row0(c), _RC)], sout[k]
                )
                if not last:
                    pltpu.async_copy(
                        tab_hbm.at[idx_v.at[c + 2]], inb[k], sin[k]
                    )

        pltpu.async_copy(tab_hbm.at[idx_v.at[0]], in0, si0)
        pltpu.async_copy(tab_hbm.at[idx_v.at[1]], in1, si1)

        chunk_pair(0, first=True, last=False)

        def body(i, carry):
            chunk_pair(2 * i, first=False, last=False)
            return carry

        lax.fori_loop(1, n_chunks // 2 - 1, body, 0)

        chunk_pair(n_chunks - 2, first=False, last=True)
        wait_out(0)
        wait_out(1)

    return tm_kernel


def kernel(instructions, gamma, beta, idx_subject, idx_label):
    S, L, T, C = instructions.shape
    B = idx_label.shape[0]

    tab = jnp.transpose(instructions, (0, 2, 1, 3)).reshape(S * T * L, C)

    base = jnp.asarray(idx_subject, jnp.int32) * (T * L)
    idxc = idx_label.astype(jnp.int32).reshape(_NW, _BSUB, _RC)
    gidx = (
        base
        + (jnp.arange(T, dtype=jnp.int32) * L)[None, :, None, None]
        + idxc[:, None, :, :]
    ).reshape(_NW, T * _BSUB, _RC)

    fn = _make_kernel(L, T, C, B)
    out = fn(tab, gidx, gamma, beta)

    return jnp.transpose(out.reshape(T, B, C), (1, 0, 2))


# speedup vs baseline: 2.2285x; 1.2935x over previous
import functools

import jax
import jax.numpy as jnp
from jax import lax
from jax.experimental import pallas as pl
from jax.experimental.pallas import tpu as pltpu
from jax.experimental.pallas import tpu_sc as plsc

_NC = 2
_NS = 16
_NW = _NC * _NS
_LANES = 16

_EPS = 1e-5
_RCA = 32
_NCA = 10
_RCB = 64
_BSUB = 2


def _rsqrt16(x):
    i = lax.bitcast_convert_type(x, jnp.int32)
    i = jnp.int32(0x5F3759DF) - lax.shift_right_logical(i, 1)
    y = lax.bitcast_convert_type(i, jnp.float32)
    half_x = 0.5 * x
    for _ in range(2):
        y = y * (1.5 - half_x * y * y)
    return y


def _lane_sum(x, perms):
    for perm in perms:
        x = x + x.at[perm].get(mode="promise_in_bounds")
    return x


def _ln_chunk(src, dst, gam_v, bet_v, perms, inv_n, n_rows):
    sub8 = src.shape[-1] // (8 * _LANES)

    def per_quad(q, carry):
        r0 = 4 * q
        r1 = r0 + 1
        r2 = r0 + 2
        r3 = r0 + 3
        zeros = jnp.zeros((_LANES,), jnp.float32)
        init = (zeros,) * 8

        def acc(u, carry3):
            a0, q0, a1, q1, a2, q2, a3, q3 = carry3
            ubase = u * (8 * _LANES)
            for k in range(8):
                col = ubase + k * _LANES
                v0 = src[r0, pl.ds(col, _LANES)]
                v1 = src[r1, pl.ds(col, _LANES)]
                v2 = src[r2, pl.ds(col, _LANES)]
                v3 = src[r3, pl.ds(col, _LANES)]
                a0 += v0
                q0 += v0 * v0
                a1 += v1
                q1 += v1 * v1
                a2 += v2
                q2 += v2 * v2
                a3 += v3
                q3 += v3 * v3
            return (a0, q0, a1, q1, a2, q2, a3, q3)

        s0, q0, s1, q1, s2, q2, s3, q3 = lax.fori_loop(0, sub8, acc, init)
        m0 = _lane_sum(s0, perms) * inv_n
        m1 = _lane_sum(s1, perms) * inv_n
        m2 = _lane_sum(s2, perms) * inv_n
        m3 = _lane_sum(s3, perms) * inv_n
        c0 = _rsqrt16(_lane_sum(q0, perms) * inv_n - m0 * m0 + _EPS)
        c1 = _rsqrt16(_lane_sum(q1, perms) * inv_n - m1 * m1 + _EPS)
        c2 = _rsqrt16(_lane_sum(q2, perms) * inv_n - m2 * m2 + _EPS)
        c3 = _rsqrt16(_lane_sum(q3, perms) * inv_n - m3 * m3 + _EPS)

        def norm(u, carry3):
            ubase = u * (8 * _LANES)
            for k in range(8):
                col = ubase + k * _LANES
                g = gam_v[pl.ds(col, _LANES)]
                b = bet_v[pl.ds(col, _LANES)]
                v0 = src[r0, pl.ds(col, _LANES)]
                v1 = src[r1, pl.ds(col, _LANES)]
                v2 = src[r2, pl.ds(col, _LANES)]
                v3 = src[r3, pl.ds(col, _LANES)]
                dst[r0, pl.ds(col, _LANES)] = (v0 - m0) * c0 * g + b
                dst[r1, pl.ds(col, _LANES)] = (v1 - m1) * c1 * g + b
                dst[r2, pl.ds(col, _LANES)] = (v2 - m2) * c2 * g + b
                dst[r3, pl.ds(col, _LANES)] = (v3 - m3) * c3 * g + b
            return carry3

        lax.fori_loop(0, sub8, norm, 0)
        return carry

    lax.fori_loop(0, n_rows // 4, per_quad, 0)


def _make_norm_kernel(SL, T, L, C):
    TL = T * L
    mesh = plsc.VectorSubcoreMesh(core_axis_name="c", subcore_axis_name="s")

    @functools.partial(
        pl.kernel,
        mesh=mesh,
        out_type=jax.ShapeDtypeStruct((TL, C), jnp.float32),
        scratch_types=[
            pltpu.VMEM((_NCA, _RCA), jnp.int32),
            pltpu.VMEM((_RCA, C), jnp.float32),
            pltpu.VMEM((_RCA, C), jnp.float32),
            pltpu.VMEM((_RCA, C), jnp.float32),
            pltpu.VMEM((_RCA, C), jnp.float32),
            pltpu.VMEM((C,), jnp.float32),
            pltpu.VMEM((C,), jnp.float32),
            pltpu.SemaphoreType.DMA,
            pltpu.SemaphoreType.DMA,
            pltpu.SemaphoreType.DMA,
            pltpu.SemaphoreType.DMA,
        ],
    )
    def norm_kernel(tab_hbm, rowidx_hbm, gamma_hbm, beta_hbm, out_hbm,
                    idx_v, in0, in1, st0, st1, gam_v, bet_v,
                    si0, si1, so0, so1):
        wid = lax.axis_index("s") * _NC + lax.axis_index("c")
        pltpu.sync_copy(rowidx_hbm.at[wid], idx_v)
        pltpu.sync_copy(gamma_hbm, gam_v)
        pltpu.sync_copy(beta_hbm, bet_v)
        lane = lax.iota(jnp.int32, _LANES)
        perms = tuple(lane ^ step for step in (8, 4, 2, 1))
        inv_n = jnp.float32(1.0 / C)

        inb = (in0, in1)
        stb = (st0, st1)
        sin = (si0, si1)
        sout = (so0, so1)

        def out_off(i):
            off = jnp.minimum((wid * _NCA + i) * _RCA, TL - _RCA)
            return pl.multiple_of(off, 8)

        def wait_in(k):
            pltpu.make_async_copy(tab_hbm.at[idx_v.at[0]], inb[k], sin[k]).wait()

        def wait_out(k):
            pltpu.make_async_copy(
                stb[k], out_hbm.at[pl.ds(0, _RCA)], sout[k]
            ).wait()

        def chunk_pair(ibase, first, last):
            for k in (0, 1):
                i = ibase + k
                wait_in(k)
                if not first:
                    wait_out(k)
                _ln_chunk(inb[k], stb[k], gam_v, bet_v, perms, inv_n, _RCA)
                pltpu.async_copy(
                    stb[k], out_hbm.at[pl.ds(out_off(i), _RCA)], sout[k]
                )
                if not last:
                    pltpu.async_copy(
                        tab_hbm.at[idx_v.at[i + 2]], inb[k], sin[k]
                    )

        pltpu.async_copy(tab_hbm.at[idx_v.at[0]], in0, si0)
        pltpu.async_copy(tab_hbm.at[idx_v.at[1]], in1, si1)
        chunk_pair(0, first=True, last=False)

        def body(ip, carry):
            chunk_pair(2 * ip, first=False, last=False)
            return carry

        lax.fori_loop(1, _NCA // 2 - 1, body, 0)
        chunk_pair(_NCA - 2, first=False, last=True)
        wait_out(0)
        wait_out(1)

    return norm_kernel


def _make_gather_kernel(TL, T, C, B):
    per_w = B // _NW
    n_chunks = T * _BSUB
    mesh = plsc.VectorSubcoreMesh(core_axis_name="c", subcore_axis_name="s")

    @functools.partial(
        pl.kernel,
        mesh=mesh,
        out_type=jax.ShapeDtypeStruct((T * B, C), jnp.float32),
        scratch_types=[
            pltpu.VMEM((n_chunks, _RCB), jnp.int32),
            pltpu.VMEM((_RCB, C), jnp.float32),
            pltpu.VMEM((_RCB, C), jnp.float32),
            pltpu.SemaphoreType.DMA,
            pltpu.SemaphoreType.DMA,
            pltpu.SemaphoreType.DMA,
            pltpu.SemaphoreType.DMA,
        ],
    )
    def gather_kernel(tab_hbm, idx_hbm, out_hbm, idx_v, b0, b1,
                      si0, si1, so0, so1):
        wid = lax.axis_index("s") * _NC + lax.axis_index("c")
        base_b = wid * per_w
        pltpu.sync_copy(idx_hbm.at[wid], idx_v)

        bufs = (b0, b1)
        sin = (si0, si1)
        sout = (so0, so1)

        def out_row0(c):
            return pl.multiple_of(
                (c // _BSUB) * B + base_b + (c % _BSUB) * _RCB, 8
            )

        def wait_in(k):
            pltpu.make_async_copy(tab_hbm.at[idx_v.at[0]], bufs[k], sin[k]).wait()

        def wait_out(k):
            pltpu.make_async_copy(
                bufs[k], out_hbm.at[pl.ds(0, _RCB)], sout[k]
            ).wait()

        def chunk_pair(cbase, first, last):
            for k in (0, 1):
                c = cbase + k
                wait_in(k)
                pltpu.async_copy(
                    bufs[k], out_hbm.at[pl.ds(out_row0(c), _RCB)], sout[k]
                )
                if not last:
                    wait_out(k)
                    pltpu.async_copy(
                        tab_hbm.at[idx_v.at[c + 2]], bufs[k], sin[k]
                    )

        pltpu.async_copy(tab_hbm.at[idx_v.at[0]], b0, si0)
        pltpu.async_copy(tab_hbm.at[idx_v.at[1]], b1, si1)
        chunk_pair(0, first=True, last=False)

        def body(ip, carry):
            chunk_pair(2 * ip, first=False, last=False)
            return carry

        lax.fori_loop(1, n_chunks // 2 - 1, body, 0)
        chunk_pair(n_chunks - 2, first=False, last=True)
        wait_out(0)
        wait_out(1)

    return gather_kernel


def kernel(instructions, gamma, beta, idx_subject, idx_label):
    S, L, T, C = instructions.shape
    B = idx_label.shape[0]
    TL = T * L

    tab = jnp.transpose(instructions, (0, 2, 1, 3)).reshape(S * T * L, C)
    base = jnp.asarray(idx_subject, jnp.int32) * TL

    off_c = jnp.minimum(
        jnp.arange(_NW * _NCA, dtype=jnp.int32) * _RCA, TL - _RCA
    )
    aidx = (
        base + off_c[:, None] + jnp.arange(_RCA, dtype=jnp.int32)[None, :]
    ).reshape(_NW, _NCA, _RCA)
    norm_fn = _make_norm_kernel(S * TL, T, L, C)
    norm_tab = norm_fn(tab, aidx, gamma, beta)

    idxc = idx_label.astype(jnp.int32).reshape(_NW, _BSUB, _RCB)
    gidx = (
        (jnp.arange(T, dtype=jnp.int32) * L)[None, :, None, None]
        + idxc[:, None, :, :]
    ).reshape(_NW, T * _BSUB, _RCB)
    gather_fn = _make_gather_kernel(TL, T, C, B)
    out = gather_fn(norm_tab, gidx)

    return jnp.transpose(out.reshape(T, B, C), (1, 0, 2))
